# Initial kernel scaffold; baseline (speedup 1.0000x reference)
import jax
import jax.numpy as jnp
from jax.experimental import pallas as pl


def kernel(x, edge_index_0, edge_index_1, W_self0, W_neigh0, b0, W_self1, W_neigh1, b1):
    raise NotImplementedError("write your pallas kernel here")


# speedup vs baseline: 4.4193x; 4.4193x over previous
import functools

import jax
import jax.numpy as jnp
from jax import lax
from jax.experimental import pallas as pl
from jax.experimental.pallas import tpu as pltpu
from jax.experimental.pallas import tpu_sc as plsc

N_NODES = 10000
N_EDGES = 320000
IN_FEATS = 128
N_HIDDEN = 128
N_CLASSES = 64

NC = 2
NS = 16
NW = NC * NS
LANES = 16

N_PAD = 10240
ROWS_PER_TILE = N_PAD // NS
K = 128
EDGES_PER_TILE = 79 * K
E_PAD = EDGES_PER_TILE * NW
PAD_DST = N_NODES + 8


@functools.lru_cache(maxsize=None)
def _make_sc_agg(F):
  mesh = plsc.VectorSubcoreMesh(
      core_axis_name="c", subcore_axis_name="s",
      num_cores=NC, num_subcores=NS)

  @functools.partial(
      pl.kernel,
      mesh=mesh,
      compiler_params=pltpu.CompilerParams(needs_layout_passes=False),
      out_type=(
          jax.ShapeDtypeStruct((NC, N_PAD, F), jnp.float32),
          jax.ShapeDtypeStruct((NC, NS * N_PAD), jnp.float32),
      ),
      scratch_types=[
          pltpu.VMEM((K,), jnp.int32),
          pltpu.VMEM((K,), jnp.int32),
          pltpu.VMEM((K, F), jnp.float32),
          pltpu.VMEM((K, F), jnp.float32),
          pltpu.VMEM((N_PAD,), jnp.float32),
          pltpu.SemaphoreType.DMA,
          pltpu.VMEM_SHARED((N_PAD, F), jnp.float32),
      ],
  )
  def sc_agg(src_hbm, dst_hbm, m_hbm, out_hbm, deg_hbm,
             srcv, dstv, rows, zbuf, degv, sem, acc):
    cid = lax.axis_index("c")
    sid = lax.axis_index("s")

    zero16 = jnp.zeros((LANES,), jnp.float32)
    one16 = jnp.ones((LANES,), jnp.float32)

    def fillz(r, carry):
      for c in range(F // LANES):
        zbuf[r, pl.ds(c * LANES, LANES)] = zero16
      return carry

    lax.fori_loop(0, K, fillz, 0)

    def filld(r, carry):
      degv[pl.ds(r * LANES, LANES)] = zero16
      return carry

    lax.fori_loop(0, N_PAD // LANES, filld, 0)

    rb = sid * ROWS_PER_TILE
    for j in range(ROWS_PER_TILE // K):
      pltpu.sync_copy(zbuf, acc.at[pl.ds(rb + j * K, K)])
    rem = ROWS_PER_TILE % K
    if rem:
      base = rb + (ROWS_PER_TILE // K) * K
      pltpu.sync_copy(zbuf.at[pl.ds(0, rem)], acc.at[pl.ds(base, rem)])

    plsc.subcore_barrier()

    wid = sid * NC + cid
    ebase = wid * EDGES_PER_TILE

    def chunk(j, carry):
      off = ebase + j * K
      pltpu.sync_copy(src_hbm.at[pl.ds(off, K)], srcv)
      pltpu.sync_copy(dst_hbm.at[pl.ds(off, K)], dstv)
      pltpu.async_copy(m_hbm.at[srcv], rows, sem).wait()
      pltpu.sync_copy(rows, acc.at[dstv], add=True)
      for c in range(K // LANES):
        idx16 = dstv[pl.ds(c * LANES, LANES)]
        plsc.addupdate_scatter(degv, [idx16], one16)
      return carry

    lax.fori_loop(0, EDGES_PER_TILE // K, chunk, 0)

    plsc.subcore_barrier()

    for j in range(ROWS_PER_TILE // K):
      pltpu.sync_copy(acc.at[pl.ds(rb + j * K, K)],
                      out_hbm.at[cid, pl.ds(rb + j * K, K)])
    if rem:
      base = rb + (ROWS_PER_TILE // K) * K
      pltpu.sync_copy(acc.at[pl.ds(base, rem)],
                      out_hbm.at[cid, pl.ds(base, rem)])
    pltpu.sync_copy(degv, deg_hbm.at[cid, pl.ds(sid * N_PAD, N_PAD)])

  return sc_agg


_TC_BLOCK = 1024


def _tc_layer_body(x_ref, p0_ref, p1_ref, d_ref,
                   ws0_ref, wn0_ref, b0_ref, ws1_ref, b1_ref,
                   h_ref, s1_ref):
  deg = jnp.maximum(jnp.sum(d_ref[...], axis=0), 1.0)[:, None]
  hn = (p0_ref[...] + p1_ref[...]) / deg
  h = (jnp.dot(x_ref[...], ws0_ref[...], preferred_element_type=jnp.float32)
       + jnp.dot(hn, wn0_ref[...], preferred_element_type=jnp.float32)
       + b0_ref[...])
  h = jnp.maximum(h, 0.0)
  h_ref[...] = h
  s1_ref[...] = (jnp.dot(h, ws1_ref[...], preferred_element_type=jnp.float32)
                 + b1_ref[...])


def _tc_layer(x, p0, p1, d, ws0, wn0, b0, ws1, b1):
  nblk = N_PAD // _TC_BLOCK
  row_spec = lambda w: pl.BlockSpec((_TC_BLOCK, w), lambda i: (i, 0))
  full_spec = lambda r, c: pl.BlockSpec((r, c), lambda i: (0, 0))
  deg_spec = pl.BlockSpec((NW, _TC_BLOCK), lambda i: (0, i))
  return pl.pallas_call(
      _tc_layer_body,
      grid=(nblk,),
      in_specs=[
          row_spec(IN_FEATS), row_spec(IN_FEATS), row_spec(IN_FEATS),
          deg_spec,
          full_spec(IN_FEATS, N_HIDDEN), full_spec(IN_FEATS, N_HIDDEN),
          full_spec(1, N_HIDDEN),
          full_spec(N_HIDDEN, N_CLASSES),
          full_spec(1, N_CLASSES),
      ],
      out_specs=[row_spec(N_HIDDEN), row_spec(N_CLASSES)],
      out_shape=[
          jax.ShapeDtypeStruct((N_PAD, N_HIDDEN), jnp.float32),
          jax.ShapeDtypeStruct((N_PAD, N_CLASSES), jnp.float32),
      ],
  )(x, p0, p1, d, ws0, wn0, b0, ws1, b1)


def _tc_combine_body(s1_ref, p0_ref, p1_ref, d_ref, wn1_ref, o_ref):
  deg = jnp.maximum(jnp.sum(d_ref[...], axis=0), 1.0)[:, None]
  hn = (p0_ref[...] + p1_ref[...]) / deg
  o_ref[...] = s1_ref[...] + jnp.dot(
      hn, wn1_ref[...], preferred_element_type=jnp.float32)


def _tc_combine(s1, p0, p1, d, wn1):
  nblk = N_PAD // _TC_BLOCK
  row_spec = lambda w: pl.BlockSpec((_TC_BLOCK, w), lambda i: (i, 0))
  deg_spec = pl.BlockSpec((NW, _TC_BLOCK), lambda i: (0, i))
  return pl.pallas_call(
      _tc_combine_body,
      grid=(nblk,),
      in_specs=[
          row_spec(N_CLASSES), row_spec(N_HIDDEN), row_spec(N_HIDDEN),
          deg_spec,
          pl.BlockSpec((N_HIDDEN, N_CLASSES), lambda i: (0, 0)),
      ],
      out_specs=row_spec(N_CLASSES),
      out_shape=jax.ShapeDtypeStruct((N_PAD, N_CLASSES), jnp.float32),
  )(s1, p0, p1, d, wn1)


def _pad_edges(edge_index):
  npad = E_PAD - N_EDGES
  src = jnp.concatenate(
      [edge_index[0].astype(jnp.int32), jnp.zeros((npad,), jnp.int32)])
  dst = jnp.concatenate(
      [edge_index[1].astype(jnp.int32),
       jnp.full((npad,), PAD_DST, jnp.int32)])
  return src, dst


@jax.jit
def kernel(x, edge_index_0, edge_index_1,
           W_self0, W_neigh0, b0, W_self1, W_neigh1, b1):
  src0, dst0 = _pad_edges(edge_index_0)
  src1, dst1 = _pad_edges(edge_index_1)

  xp = jnp.pad(x, ((0, N_PAD - N_NODES), (0, 0)))
  p_l0, d_l0 = _make_sc_agg(IN_FEATS)(src0, dst0, xp)
  h, s1 = _tc_layer(
      xp,
      p_l0[0], p_l0[1], d_l0.reshape(NW, N_PAD),
      W_self0, W_neigh0, b0.reshape(1, -1),
      W_self1, b1.reshape(1, -1))

  p_l1, d_l1 = _make_sc_agg(N_HIDDEN)(src1, dst1, h)
  out = _tc_combine(s1, p_l1[0], p_l1[1], d_l1.reshape(NW, N_PAD), W_neigh1)
  return out[:N_NODES]
